# baseline (speedup 1.0000x reference)
import jax
import jax.numpy as jnp
from jax import lax
from jax.experimental import pallas as pl
from jax.experimental.pallas import tpu as pltpu
from jax.experimental.pallas import tpu_sc as plsc

B = 16384
N_DENSE = 13
NF = 26
VOCAB = 100000
ED = 16
NCOL = N_DENSE + NF

VOCAB_PAD = 98 * 1024
S_LEN = NF * VOCAB_PAD

NC, NS, L = 2, 16, 16
NW = NC * NS
RPW = B // NW
NIDX = RPW * NF
SEG = 128
NSEG_F = RPW // SEG
GRP = RPW // L


def _rowsum_body(x_ref, o_ref):
  o_ref[...] = jnp.sum(x_ref[0], axis=0)


RS_SPLIT = 7


def _rowsum(emb_t):
  vb = VOCAB_PAD // RS_SPLIT
  return pl.pallas_call(
      _rowsum_body,
      grid=(NF, RS_SPLIT),
      in_specs=[pl.BlockSpec((1, ED, vb), lambda f, v: (f, 0, v))],
      out_specs=pl.BlockSpec((vb,), lambda f, v: (f * RS_SPLIT + v,)),
      out_shape=jax.ShapeDtypeStruct((S_LEN,), jnp.float32),
  )(emb_t)


def _sc_build_body(inpt_hbm, w_hbm, bias_hbm, ids_hbm, part_hbm,
                   inp_v, idx_v, outb, wv, bv):
  wid = lax.axis_index("s") * NC + lax.axis_index("c")
  base = wid * RPW
  pltpu.sync_copy(inpt_hbm.at[:, pl.ds(base, RPW)], inp_v)
  pltpu.sync_copy(w_hbm, wv)
  pltpu.sync_copy(bias_hbm, bv)

  def build_field(f, carry):
    off = f * VOCAB_PAD

    def chunk4(c, c2):
      for u in range(4):
        o = (c * 4 + u) * L
        idx_v[pl.ds(f * RPW + o, L)] = (
            inp_v[N_DENSE + f, pl.ds(o, L)].astype(jnp.int32) + off)
      return c2

    lax.fori_loop(0, GRP // 4, chunk4, 0)
    return carry

  lax.fori_loop(0, NF, build_field, 0)
  pltpu.sync_copy(idx_v, ids_hbm.at[pl.ds(wid * NIDX, NIDX)])

  bvec = bv[...]

  def dense_stage(g, c):
    out_vec = bvec
    for k in range(N_DENSE):
      out_vec = out_vec + inp_v[k, pl.ds(g * L, L)] * wv[k, :]
    outb[pl.ds(g * L, L)] = out_vec
    return c

  lax.fori_loop(0, GRP, dense_stage, 0)
  pltpu.sync_copy(outb, part_hbm.at[pl.ds(base, RPW)])


def _sc_gather_body(ids_hbm, s_hbm, part_hbm, out_hbm,
                    idx_v, gbuf, outb, sem):
  wid = lax.axis_index("s") * NC + lax.axis_index("c")
  base = wid * RPW
  pltpu.sync_copy(ids_hbm.at[pl.ds(wid * NIDX, NIDX)], idx_v)

  def issue(m, c):
    o = m * SEG
    pltpu.async_copy(s_hbm.at[idx_v.at[pl.ds(o, SEG)]],
                     gbuf.at[pl.ds(o, SEG)], sem)
    return c

  lax.fori_loop(0, NIDX // SEG, issue, 0)
  pltpu.sync_copy(part_hbm.at[pl.ds(base, RPW)], outb)
  pltpu.make_async_copy(s_hbm.at[pl.ds(0, NIDX)], gbuf, sem).wait()

  def grp_stage(g, c):
    out_vec = outb[pl.ds(g * L, L)]
    for p in range(NF):
      out_vec = out_vec + gbuf[pl.ds(p * RPW + g * L, L)]
    outb[pl.ds(g * L, L)] = out_vec
    return c

  lax.fori_loop(0, GRP, grp_stage, 0)
  pltpu.sync_copy(outb, out_hbm.at[pl.ds(base, RPW)])


def kernel(inputs, emb_tables, dense_weight, bias):
  s_flat = _rowsum(emb_tables.transpose(0, 2, 1))
  w_bcast = jnp.broadcast_to(dense_weight, (N_DENSE, L))
  bias_vec = jnp.broadcast_to(bias, (L,))

  mesh = plsc.VectorSubcoreMesh(core_axis_name="c", subcore_axis_name="s")
  cparams = pltpu.CompilerParams(
      needs_layout_passes=False, use_tc_tiling_on_sc=False)

  ids, partial = pl.kernel(
      _sc_build_body,
      out_type=(jax.ShapeDtypeStruct((NW * NIDX,), jnp.int32),
                jax.ShapeDtypeStruct((B,), jnp.float32)),
      mesh=mesh,
      compiler_params=cparams,
      scratch_types=[
          pltpu.VMEM((NCOL, RPW), jnp.float32),
          pltpu.VMEM((NIDX,), jnp.int32),
          pltpu.VMEM((RPW,), jnp.float32),
          pltpu.VMEM((N_DENSE, L), jnp.float32),
          pltpu.VMEM((L,), jnp.float32),
      ],
  )(inputs.T, w_bcast, bias_vec)

  out = pl.kernel(
      _sc_gather_body,
      out_type=jax.ShapeDtypeStruct((B,), jnp.float32),
      mesh=mesh,
      compiler_params=cparams,
      scratch_types=[
          pltpu.VMEM((NIDX,), jnp.int32),
          pltpu.VMEM((NIDX,), jnp.float32),
          pltpu.VMEM((RPW,), jnp.float32),
          pltpu.SemaphoreType.DMA,
      ],
  )(ids, s_flat, partial)
  return out[:, None]


# speedup vs baseline: 1.7815x; 1.7815x over previous
import jax
import jax.numpy as jnp
from jax import lax
from jax.experimental import pallas as pl
from jax.experimental.pallas import tpu as pltpu
from jax.experimental.pallas import tpu_sc as plsc

B = 16384
N_DENSE = 13
NF = 26
VOCAB = 100000
ED = 16
NCOL = N_DENSE + NF

VOCAB_PAD = 98 * 1024
S_LEN = NF * VOCAB_PAD

NC, NS, L = 2, 16, 16
NW = NC * NS
RPW = B // NW
NIDX = RPW * NF
SEG = 128
NSEG_F = RPW // SEG
GRP = RPW // L


def _rowsum_body(x_ref, o_ref):
  o_ref[...] = jnp.sum(x_ref[...], axis=1).reshape(-1)


FB = 2


def _rowsum(emb_t):
  return pl.pallas_call(
      _rowsum_body,
      grid=(NF // FB,),
      in_specs=[pl.BlockSpec((FB, ED, VOCAB_PAD), lambda f: (f, 0, 0))],
      out_specs=pl.BlockSpec((FB * VOCAB_PAD,), lambda f: (f,)),
      out_shape=jax.ShapeDtypeStruct((S_LEN,), jnp.float32),
  )(emb_t)


def _sc_build_body(inpt_hbm, w_hbm, bias_hbm, ids_hbm, part_hbm,
                   inp_v, idx_v, outb, wv, bv):
  wid = lax.axis_index("s") * NC + lax.axis_index("c")
  base = wid * RPW
  pltpu.sync_copy(inpt_hbm.at[:, pl.ds(base, RPW)], inp_v)
  pltpu.sync_copy(w_hbm, wv)
  pltpu.sync_copy(bias_hbm, bv)

  def build_field(f, carry):
    off = f * VOCAB_PAD

    def chunk4(c, c2):
      for u in range(4):
        o = (c * 4 + u) * L
        idx_v[pl.ds(f * RPW + o, L)] = (
            inp_v[N_DENSE + f, pl.ds(o, L)].astype(jnp.int32) + off)
      return c2

    lax.fori_loop(0, GRP // 4, chunk4, 0)
    return carry

  lax.fori_loop(0, NF, build_field, 0)
  pltpu.sync_copy(idx_v, ids_hbm.at[pl.ds(wid * NIDX, NIDX)])

  bvec = bv[...]

  def dense_stage(g, c):
    out_vec = bvec
    for k in range(N_DENSE):
      out_vec = out_vec + inp_v[k, pl.ds(g * L, L)] * wv[k, :]
    outb[pl.ds(g * L, L)] = out_vec
    return c

  lax.fori_loop(0, GRP, dense_stage, 0)
  pltpu.sync_copy(outb, part_hbm.at[pl.ds(base, RPW)])


def _sc_gather_body(ids_hbm, s_hbm, part_hbm, out_hbm,
                    idx_v, gbuf, outb, sem):
  wid = lax.axis_index("s") * NC + lax.axis_index("c")
  base = wid * RPW
  pltpu.sync_copy(ids_hbm.at[pl.ds(wid * NIDX, NIDX)], idx_v)

  def issue(m, c):
    o = m * SEG
    pltpu.async_copy(s_hbm.at[idx_v.at[pl.ds(o, SEG)]],
                     gbuf.at[pl.ds(o, SEG)], sem)
    return c

  lax.fori_loop(0, NIDX // SEG, issue, 0)
  pltpu.sync_copy(part_hbm.at[pl.ds(base, RPW)], outb)
  pltpu.make_async_copy(s_hbm.at[pl.ds(0, NIDX)], gbuf, sem).wait()

  def grp_stage(g, c):
    out_vec = outb[pl.ds(g * L, L)]
    for p in range(NF):
      out_vec = out_vec + gbuf[pl.ds(p * RPW + g * L, L)]
    outb[pl.ds(g * L, L)] = out_vec
    return c

  lax.fori_loop(0, GRP, grp_stage, 0)
  pltpu.sync_copy(outb, out_hbm.at[pl.ds(base, RPW)])


def kernel(inputs, emb_tables, dense_weight, bias):
  s_flat = _rowsum(emb_tables.transpose(0, 2, 1))
  w_bcast = jnp.broadcast_to(dense_weight, (N_DENSE, L))
  bias_vec = jnp.broadcast_to(bias, (L,))

  mesh = plsc.VectorSubcoreMesh(core_axis_name="c", subcore_axis_name="s")
  cparams = pltpu.CompilerParams(
      needs_layout_passes=False, use_tc_tiling_on_sc=False)

  ids, partial = pl.kernel(
      _sc_build_body,
      out_type=(jax.ShapeDtypeStruct((NW * NIDX,), jnp.int32),
                jax.ShapeDtypeStruct((B,), jnp.float32)),
      mesh=mesh,
      compiler_params=cparams,
      scratch_types=[
          pltpu.VMEM((NCOL, RPW), jnp.float32),
          pltpu.VMEM((NIDX,), jnp.int32),
          pltpu.VMEM((RPW,), jnp.float32),
          pltpu.VMEM((N_DENSE, L), jnp.float32),
          pltpu.VMEM((L,), jnp.float32),
      ],
  )(inputs.T, w_bcast, bias_vec)

  out = pl.kernel(
      _sc_gather_body,
      out_type=jax.ShapeDtypeStruct((B,), jnp.float32),
      mesh=mesh,
      compiler_params=cparams,
      scratch_types=[
          pltpu.VMEM((NIDX,), jnp.int32),
          pltpu.VMEM((NIDX,), jnp.float32),
          pltpu.VMEM((RPW,), jnp.float32),
          pltpu.SemaphoreType.DMA,
      ],
  )(ids, s_flat, partial)
  return out[:, None]
